# baseline (speedup 1.0000x reference)
import functools

import jax
import jax.numpy as jnp
from jax import lax
from jax.experimental import pallas as pl
from jax.experimental.pallas import tpu as pltpu
from jax.experimental.pallas import tpu_sc as plsc

_B, _L = 4096, 200
_D = 32
_DPAD = 128
_N = _B * _L


def _make_gather(chunk: int):
    info = plsc.get_sparse_core_info()
    nc, ns = info.num_cores, info.num_subcores
    nw = nc * ns
    per_w = _N // nw
    n_chunks = per_w // chunk
    assert per_w % chunk == 0

    mesh = plsc.VectorSubcoreMesh(core_axis_name="c", subcore_axis_name="s")

    @functools.partial(
        pl.kernel,
        mesh=mesh,
        out_type=jax.ShapeDtypeStruct((_B, _L, _D), jnp.float32),
        scratch_types=[
            pltpu.VMEM((chunk,), jnp.int32),
            pltpu.VMEM((chunk, _DPAD), jnp.float32),
            pltpu.VMEM((chunk, _D), jnp.float32),
            pltpu.SemaphoreType.DMA,
        ],
    )
    def k(idx_hbm, table_hbm, out_hbm, idx_v, big_v, out_v, sem):
        wid = lax.axis_index("s") * nc + lax.axis_index("c")
        base = wid * per_w
        ov = out_hbm.reshape(_N, _D)

        def body(i, carry):
            off = base + i * chunk
            pltpu.sync_copy(idx_hbm.at[pl.ds(off, chunk)], idx_v)
            pltpu.async_copy(table_hbm.at[idx_v], big_v, sem).wait()

            def sel(r, c):
                out_v[r, pl.ds(0, 16)] = big_v[r, pl.ds(0, 16)]
                out_v[r, pl.ds(16, 16)] = big_v[r, pl.ds(16, 16)]
                return c

            lax.fori_loop(0, chunk, sel, 0)
            pltpu.sync_copy(out_v, ov.at[pl.ds(off, chunk)])
            return carry

        lax.fori_loop(0, n_chunks, body, 0)

    return k


_gather = _make_gather(chunk=256)


@jax.jit
def kernel(x, table):
    idx = x.reshape(-1).astype(jnp.int32)
    tq = jnp.pad(table, ((0, 0), (0, _DPAD - _D)))
    out = _gather(idx, tq)
    return out
